# baseline (device time: 94975 ns/iter reference)
import jax
import jax.numpy as jnp
from jax import lax
from jax.experimental import pallas as pl
from jax.experimental.pallas import tpu as pltpu

N_DEV = 8
NSUB = 4


def _ring(t):
    return jnp.where(t < 4, t, 11 - t)


def kernel(x, w_mat, scale_x, scale_w):
    m, k = x.shape
    _, n = w_mat.shape
    chunk = m // N_DEV
    nh = n // 2

    def body(x_ref, w_ref, sx_ref, sw_ref, out_ref,
             x16, w16, comm_cw, comm_ccw,
             send_cw, recv_cw, send_ccw, recv_ccw):
        p = lax.axis_index("i")
        r = _ring(p)
        right = _ring((r + 1) % N_DEV)
        left = _ring((r + N_DEV - 1) % N_DEV)

        bar = pltpu.get_barrier_semaphore()
        for nbr in (left, right):
            pl.semaphore_signal(bar, inc=1, device_id=(nbr,),
                                device_id_type=pl.DeviceIdType.MESH)
        pl.semaphore_wait(bar, 2)

        x16[...] = x_ref[...].astype(jnp.bfloat16)
        w16[...] = w_ref[...].astype(jnp.bfloat16)

        def partial(row, hi):
            a = x16[pl.ds(row, chunk), :]
            b = w16[:, nh:] if hi else w16[:, :nh]
            return jnp.dot(a, b, preferred_element_type=jnp.float32)

        sub = nh // NSUB

        def make_rdma(comm, sems_s, sems_r, h, j, tgt):
            cs = j * sub
            return pltpu.make_async_remote_copy(
                src_ref=comm.at[h, :, pl.ds(cs, sub)],
                dst_ref=comm.at[h + 1, :, pl.ds(cs, sub)],
                send_sem=sems_s.at[h, j], recv_sem=sems_r.at[h + 1, j],
                device_id=(tgt,), device_id_type=pl.DeviceIdType.MESH)

        comm_cw[0, :, :] = partial(
            _ring((r + N_DEV - 1) % N_DEV) * chunk, hi=False).astype(jnp.bfloat16)
        comm_ccw[0, :, :] = partial(
            _ring((r + 1) % N_DEV) * chunk, hi=True).astype(jnp.bfloat16)
        prev = []
        for j in range(NSUB):
            rd_cw = make_rdma(comm_cw, send_cw, recv_cw, 0, j, right)
            rd_ccw = make_rdma(comm_ccw, send_ccw, recv_ccw, 0, j, left)
            rd_cw.start()
            rd_ccw.start()
            prev.append((rd_cw, rd_ccw))

        for h in range(1, N_DEV - 1):
            pcw = partial(_ring((r + N_DEV - 1 - h) % N_DEV) * chunk, hi=False)
            pccw = partial(_ring((r + 1 + h) % N_DEV) * chunk, hi=True)
            cur = []
            for j in range(NSUB):
                cs = j * sub
                rd_cw, rd_ccw = prev[j]
                rd_cw.wait()
                comm_cw[h, :, pl.ds(cs, sub)] = (
                    comm_cw[h, :, pl.ds(cs, sub)].astype(jnp.float32)
                    + pcw[:, cs:cs + sub]).astype(jnp.bfloat16)
                nrd_cw = make_rdma(comm_cw, send_cw, recv_cw, h, j, right)
                nrd_cw.start()
                rd_ccw.wait()
                comm_ccw[h, :, pl.ds(cs, sub)] = (
                    comm_ccw[h, :, pl.ds(cs, sub)].astype(jnp.float32)
                    + pccw[:, cs:cs + sub]).astype(jnp.bfloat16)
                nrd_ccw = make_rdma(comm_ccw, send_ccw, recv_ccw, h, j, left)
                nrd_ccw.start()
                cur.append((nrd_cw, nrd_ccw))
            prev = cur

        s = sx_ref[0] * sw_ref[0]
        own = p * chunk
        p_own_cw = partial(own, hi=False)
        p_own_ccw = partial(own, hi=True)
        for rd_cw, rd_ccw in prev:
            rd_cw.wait()
            rd_ccw.wait()
        acc_cw = comm_cw[N_DEV - 1, :, :].astype(jnp.float32) + p_own_cw
        acc_ccw = comm_ccw[N_DEV - 1, :, :].astype(jnp.float32) + p_own_ccw
        out_ref[:, :nh] = jnp.maximum(acc_cw * s, 0.0)
        out_ref[:, nh:] = jnp.maximum(acc_ccw * s, 0.0)

    return pl.pallas_call(
        body,
        out_shape=jax.ShapeDtypeStruct((chunk, n), jnp.float32),
        in_specs=[
            pl.BlockSpec(memory_space=pltpu.VMEM),
            pl.BlockSpec(memory_space=pltpu.VMEM),
            pl.BlockSpec(memory_space=pltpu.SMEM),
            pl.BlockSpec(memory_space=pltpu.SMEM),
        ],
        out_specs=pl.BlockSpec(memory_space=pltpu.VMEM),
        scratch_shapes=[
            pltpu.VMEM((m, k), jnp.bfloat16),
            pltpu.VMEM((k, n), jnp.bfloat16),
            pltpu.VMEM((N_DEV, chunk, nh), jnp.bfloat16),
            pltpu.VMEM((N_DEV, chunk, nh), jnp.bfloat16),
            pltpu.SemaphoreType.DMA((N_DEV, NSUB)),
            pltpu.SemaphoreType.DMA((N_DEV, NSUB)),
            pltpu.SemaphoreType.DMA((N_DEV, NSUB)),
            pltpu.SemaphoreType.DMA((N_DEV, NSUB)),
        ],
        compiler_params=pltpu.CompilerParams(collective_id=0),
    )(x, w_mat, scale_x, scale_w)


# device time: 93814 ns/iter; 1.0124x vs baseline; 1.0124x over previous
import jax
import jax.numpy as jnp
from jax import lax
from jax.experimental import pallas as pl
from jax.experimental.pallas import tpu as pltpu

N_DEV = 8
NSUB = 2


def _ring(t):
    return jnp.where(t < 4, t, 11 - t)


def kernel(x, w_mat, scale_x, scale_w):
    m, k = x.shape
    _, n = w_mat.shape
    chunk = m // N_DEV
    nh = n // 2

    def body(x_ref, w_ref, sx_ref, sw_ref, out_ref,
             x16, w16, comm_cw, comm_ccw,
             send_cw, recv_cw, send_ccw, recv_ccw):
        p = lax.axis_index("i")
        r = _ring(p)
        right = _ring((r + 1) % N_DEV)
        left = _ring((r + N_DEV - 1) % N_DEV)

        bar = pltpu.get_barrier_semaphore()
        for nbr in (left, right):
            pl.semaphore_signal(bar, inc=1, device_id=(nbr,),
                                device_id_type=pl.DeviceIdType.MESH)
        pl.semaphore_wait(bar, 2)

        x16[...] = x_ref[...].astype(x16.dtype)
        w16[...] = w_ref[...].astype(w16.dtype)

        def partial(row, hi):
            a = x16[pl.ds(row, chunk), :]
            b = w16[:, nh:] if hi else w16[:, :nh]
            return jnp.dot(a, b, preferred_element_type=jnp.float32)

        sub = nh // NSUB

        def make_rdma(comm, sems_s, sems_r, h, j, tgt):
            cs = j * sub
            return pltpu.make_async_remote_copy(
                src_ref=comm.at[h, :, pl.ds(cs, sub)],
                dst_ref=comm.at[h + 1, :, pl.ds(cs, sub)],
                send_sem=sems_s.at[h, j], recv_sem=sems_r.at[h + 1, j],
                device_id=(tgt,), device_id_type=pl.DeviceIdType.MESH)

        comm_cw[0, :, :] = partial(
            _ring((r + N_DEV - 1) % N_DEV) * chunk, hi=False).astype(jnp.bfloat16)
        comm_ccw[0, :, :] = partial(
            _ring((r + 1) % N_DEV) * chunk, hi=True).astype(jnp.bfloat16)
        prev = []
        for j in range(NSUB):
            rd_cw = make_rdma(comm_cw, send_cw, recv_cw, 0, j, right)
            rd_ccw = make_rdma(comm_ccw, send_ccw, recv_ccw, 0, j, left)
            rd_cw.start()
            rd_ccw.start()
            prev.append((rd_cw, rd_ccw))

        for h in range(1, N_DEV - 1):
            pcw = partial(_ring((r + N_DEV - 1 - h) % N_DEV) * chunk, hi=False)
            pccw = partial(_ring((r + 1 + h) % N_DEV) * chunk, hi=True)
            cur = []
            for j in range(NSUB):
                cs = j * sub
                rd_cw, rd_ccw = prev[j]
                rd_cw.wait()
                comm_cw[h, :, pl.ds(cs, sub)] = (
                    comm_cw[h, :, pl.ds(cs, sub)].astype(jnp.float32)
                    + pcw[:, cs:cs + sub]).astype(jnp.bfloat16)
                nrd_cw = make_rdma(comm_cw, send_cw, recv_cw, h, j, right)
                nrd_cw.start()
                rd_ccw.wait()
                comm_ccw[h, :, pl.ds(cs, sub)] = (
                    comm_ccw[h, :, pl.ds(cs, sub)].astype(jnp.float32)
                    + pccw[:, cs:cs + sub]).astype(jnp.bfloat16)
                nrd_ccw = make_rdma(comm_ccw, send_ccw, recv_ccw, h, j, left)
                nrd_ccw.start()
                cur.append((nrd_cw, nrd_ccw))
            prev = cur

        s = sx_ref[0] * sw_ref[0]
        own = p * chunk
        p_own_cw = partial(own, hi=False)
        p_own_ccw = partial(own, hi=True)
        for rd_cw, rd_ccw in prev:
            rd_cw.wait()
            rd_ccw.wait()
        acc_cw = comm_cw[N_DEV - 1, :, :].astype(jnp.float32) + p_own_cw
        acc_ccw = comm_ccw[N_DEV - 1, :, :].astype(jnp.float32) + p_own_ccw
        out_ref[:, :nh] = jnp.maximum(acc_cw * s, 0.0)
        out_ref[:, nh:] = jnp.maximum(acc_ccw * s, 0.0)

    return pl.pallas_call(
        body,
        out_shape=jax.ShapeDtypeStruct((chunk, n), jnp.float32),
        in_specs=[
            pl.BlockSpec(memory_space=pltpu.VMEM),
            pl.BlockSpec(memory_space=pltpu.VMEM),
            pl.BlockSpec(memory_space=pltpu.SMEM),
            pl.BlockSpec(memory_space=pltpu.SMEM),
        ],
        out_specs=pl.BlockSpec(memory_space=pltpu.VMEM),
        scratch_shapes=[
            pltpu.VMEM((m, k), jnp.float8_e4m3fn),
            pltpu.VMEM((k, n), jnp.float8_e5m2),
            pltpu.VMEM((N_DEV, chunk, nh), jnp.bfloat16),
            pltpu.VMEM((N_DEV, chunk, nh), jnp.bfloat16),
            pltpu.SemaphoreType.DMA((N_DEV, NSUB)),
            pltpu.SemaphoreType.DMA((N_DEV, NSUB)),
            pltpu.SemaphoreType.DMA((N_DEV, NSUB)),
            pltpu.SemaphoreType.DMA((N_DEV, NSUB)),
        ],
        compiler_params=pltpu.CompilerParams(collective_id=0),
    )(x, w_mat, scale_x, scale_w)
